# baseline (speedup 1.0000x reference)
import math
import functools

import jax
import jax.numpy as jnp
from jax import lax
from jax.experimental import pallas as pl
from jax.experimental.pallas import tpu as pltpu
from jax.experimental.pallas import tpu_sc as plsc

NC = 2
NS = 16
NW = NC * NS
LANES = 16


def _make_kernel(B, S, D, CH):
    SPW = S // NW
    T = (B * SPW) // CH
    CPB = SPW // CH
    scale = 1.0 / math.sqrt(D)
    mesh = plsc.VectorSubcoreMesh(
        core_axis_name="c", subcore_axis_name="s",
        num_cores=NC, num_subcores=NS)

    @functools.partial(
        pl.kernel,
        out_type=jax.ShapeDtypeStruct((B * S, D), jnp.float32),
        mesh=mesh,
        scratch_types=[
            pltpu.VMEM((B * SPW,), jnp.int32),
            pltpu.VMEM((SPW, D), jnp.float32),
            pltpu.VMEM((CH, D), jnp.float32),
            pltpu.VMEM((CH, D), jnp.float32),
            pltpu.SemaphoreType.DMA,
            pltpu.SemaphoreType.DMA,
            pltpu.SemaphoreType.DMA,
            pltpu.SemaphoreType.DMA,
        ],
    )
    def k(x_hbm, tok_hbm, pos_hbm, out_hbm,
          idx_v, pos_v, rows0, rows1, g0, g1, w0, w1):
        wid = lax.axis_index("s") * NC + lax.axis_index("c")
        s_base = wid * SPW
        rows = (rows0, rows1)
        gsem = (g0, g1)
        wsem = (w0, w1)

        for b in range(B):
            pltpu.sync_copy(x_hbm.at[pl.ds(b * S + s_base, SPW)],
                            idx_v.at[pl.ds(b * SPW, SPW)])
        pltpu.sync_copy(pos_hbm.at[pl.ds(s_base, SPW), :], pos_v)

        def out_base(t):
            return (t // CPB) * S + s_base + (t % CPB) * CH

        def start_gather(t, buf):
            pltpu.async_copy(tok_hbm.at[idx_v.at[pl.ds(t * CH, CH)]],
                             rows[buf], gsem[buf])

        start_gather(0, 0)

        @pl.loop(0, T, step=2)
        def _pair(t0):
            for kk in range(2):
                t = t0 + kk
                buf = kk
                nxt_buf = 1 - kk

                @pl.when(t + 1 < T)
                def _():
                    @pl.when(t >= 1)
                    def _():
                        pltpu.make_async_copy(
                            rows[nxt_buf],
                            out_hbm.at[pl.ds(0, CH), :],
                            wsem[nxt_buf]).wait()
                    start_gather(t + 1, nxt_buf)

                pltpu.make_async_copy(
                    tok_hbm.at[idx_v.at[pl.ds(0, CH)]],
                    rows[buf], gsem[buf]).wait()

                s_off = (t % CPB) * CH
                rbuf = rows[buf]

                @pl.loop(0, CH)
                def _row(r):
                    pr = s_off + r

                    @pl.loop(0, D // LANES, unroll=4)
                    def _col(c):
                        cs = c * LANES
                        rbuf[r, pl.ds(cs, LANES)] = (
                            rbuf[r, pl.ds(cs, LANES)] * scale
                            + pos_v[pr, pl.ds(cs, LANES)])

                pltpu.async_copy(rbuf, out_hbm.at[pl.ds(out_base(t), CH), :],
                                 wsem[buf])

        for buf in range(2):
            pltpu.make_async_copy(rows[buf], out_hbm.at[pl.ds(0, CH), :],
                                  wsem[buf]).wait()

    return k


@jax.jit
def kernel(x, tok_emb, pos_emb):
    B, S = x.shape
    D = tok_emb.shape[1]
    xf = x.reshape(B * S).astype(jnp.int32)
    out = _make_kernel(B, S, D, CH=16)(xf, tok_emb, pos_emb)
    return out.reshape(B, S, D)


# speedup vs baseline: 2.3398x; 2.3398x over previous
import math
import functools

import jax
import jax.numpy as jnp
from jax import lax
from jax.experimental import pallas as pl
from jax.experimental.pallas import tpu as pltpu
from jax.experimental.pallas import tpu_sc as plsc

NC = 2
NS = 16
NW = NC * NS
LANES = 16


def _make_kernel(B, S, D, CH):
    SPW = S // NW
    T = (B * SPW) // CH
    CPB = SPW // CH
    scale = 1.0 / math.sqrt(D)
    mesh = plsc.VectorSubcoreMesh(
        core_axis_name="c", subcore_axis_name="s",
        num_cores=NC, num_subcores=NS)

    @functools.partial(
        pl.kernel,
        out_type=jax.ShapeDtypeStruct((B * S, D), jnp.float32),
        mesh=mesh,
        scratch_types=[
            pltpu.VMEM((B * SPW,), jnp.int32),
            pltpu.VMEM((SPW, D), jnp.float32),
            pltpu.VMEM((CH, D), jnp.float32),
            pltpu.VMEM((CH, D), jnp.float32),
            pltpu.SemaphoreType.DMA,
            pltpu.SemaphoreType.DMA,
            pltpu.SemaphoreType.DMA,
            pltpu.SemaphoreType.DMA,
        ],
    )
    def k(x_hbm, tok_hbm, pos_hbm, out_hbm,
          idx_v, pos_v, rows0, rows1, g0, g1, w0, w1):
        wid = lax.axis_index("s") * NC + lax.axis_index("c")
        s_base = wid * SPW
        rows = (rows0, rows1)
        gsem = (g0, g1)
        wsem = (w0, w1)

        for b in range(B):
            pltpu.sync_copy(x_hbm.at[pl.ds(b * S + s_base, SPW)],
                            idx_v.at[pl.ds(b * SPW, SPW)])
        pltpu.sync_copy(pos_hbm.at[pl.ds(s_base, SPW), :], pos_v)

        def out_base(t):
            return (t // CPB) * S + s_base + (t % CPB) * CH

        def start_gather(t, buf):
            pltpu.async_copy(tok_hbm.at[idx_v.at[pl.ds(t * CH, CH)]],
                             rows[buf], gsem[buf])

        start_gather(0, 0)

        @pl.loop(0, T, step=2)
        def _pair(t0):
            for kk in range(2):
                t = t0 + kk
                buf = kk
                nxt_buf = 1 - kk

                @pl.when(t + 1 < T)
                def _():
                    @pl.when(t >= 1)
                    def _():
                        pltpu.make_async_copy(
                            rows[nxt_buf],
                            out_hbm.at[pl.ds(0, CH), :],
                            wsem[nxt_buf]).wait()
                    start_gather(t + 1, nxt_buf)

                pltpu.make_async_copy(
                    tok_hbm.at[idx_v.at[pl.ds(0, CH)]],
                    rows[buf], gsem[buf]).wait()

                s_off = (t % CPB) * CH
                rbuf = rows[buf]
                VPR = D // LANES

                @plsc.parallel_loop(0, CH * VPR, unroll=8)
                def _v(i):
                    r = i // VPR
                    cs = (i % VPR) * LANES
                    rbuf[r, pl.ds(cs, LANES)] = (
                        rbuf[r, pl.ds(cs, LANES)] * scale
                        + pos_v[s_off + r, pl.ds(cs, LANES)])

                pltpu.async_copy(rbuf, out_hbm.at[pl.ds(out_base(t), CH), :],
                                 wsem[buf])

        for buf in range(2):
            pltpu.make_async_copy(rows[buf], out_hbm.at[pl.ds(0, CH), :],
                                  wsem[buf]).wait()

    return k


@jax.jit
def kernel(x, tok_emb, pos_emb):
    B, S = x.shape
    D = tok_emb.shape[1]
    xf = x.reshape(B * S).astype(jnp.int32)
    out = _make_kernel(B, S, D, CH=16)(xf, tok_emb, pos_emb)
    return out.reshape(B, S, D)


# speedup vs baseline: 2.4334x; 1.0400x over previous
import math
import functools

import jax
import jax.numpy as jnp
from jax import lax
from jax.experimental import pallas as pl
from jax.experimental.pallas import tpu as pltpu
from jax.experimental.pallas import tpu_sc as plsc

NC = 2
NS = 16
NW = NC * NS
LANES = 16


def _make_kernel(B, S, D, CH):
    SPW = S // NW
    HALF = SPW // 2
    CPH = HALF // CH
    T = (B * SPW) // CH
    PER_HALF = B * CPH
    scale = 1.0 / math.sqrt(D)
    VPR = D // LANES
    mesh = plsc.VectorSubcoreMesh(
        core_axis_name="c", subcore_axis_name="s",
        num_cores=NC, num_subcores=NS)

    @functools.partial(
        pl.kernel,
        out_type=jax.ShapeDtypeStruct((B * S, D), jnp.float32),
        mesh=mesh,
        scratch_types=[
            pltpu.VMEM((B * SPW,), jnp.int32),
            pltpu.VMEM((HALF, D), jnp.float32),
            pltpu.VMEM((CH, D), jnp.float32),
            pltpu.VMEM((CH, D), jnp.float32),
            pltpu.VMEM((CH, D), jnp.float32),
            pltpu.VMEM((CH, D), jnp.float32),
            pltpu.SemaphoreType.DMA,
            pltpu.SemaphoreType.DMA,
            pltpu.SemaphoreType.DMA,
            pltpu.SemaphoreType.DMA,
        ],
    )
    def k(x_hbm, tok_hbm, pos_hbm, out_hbm,
          idx_v, pos_v, ga, gb, oa, ob, g0, g1, w0, w1):
        wid = lax.axis_index("s") * NC + lax.axis_index("c")
        s_base = wid * SPW
        gbuf = (ga, gb)
        obuf = (oa, ob)
        gsem = (g0, g1)
        wsem = (w0, w1)

        for b in range(B):
            pltpu.sync_copy(x_hbm.at[pl.ds(b * S + s_base, SPW)],
                            idx_v.at[pl.ds(b * SPW, SPW)])
        pltpu.sync_copy(pos_hbm.at[pl.ds(s_base, HALF), :], pos_v)

        def decode(t):
            half = t // PER_HALF
            u = t % PER_HALF
            b = u % B
            sc = u // B
            s_off = half * HALF + sc * CH
            return b, s_off, sc * CH

        def start_gather(t, buf):
            b, s_off, _ = decode(t)
            pltpu.async_copy(
                tok_hbm.at[idx_v.at[pl.ds(b * SPW + s_off, CH)]],
                gbuf[buf], gsem[buf])

        start_gather(0, 0)
        start_gather(1, 1)

        @pl.loop(0, T, step=2)
        def _pair(t0):
            for kk in range(2):
                t = t0 + kk
                buf = kk

                @pl.when(t == PER_HALF)
                def _():
                    pltpu.sync_copy(
                        pos_hbm.at[pl.ds(s_base + HALF, HALF), :], pos_v)

                pltpu.make_async_copy(
                    tok_hbm.at[idx_v.at[pl.ds(0, CH)]],
                    gbuf[buf], gsem[buf]).wait()

                @pl.when(t >= 2)
                def _():
                    pltpu.make_async_copy(
                        obuf[buf], out_hbm.at[pl.ds(0, CH), :],
                        wsem[buf]).wait()

                b, s_off, p_off = decode(t)
                g = gbuf[buf]
                o = obuf[buf]

                @plsc.parallel_loop(0, CH * VPR, unroll=8)
                def _v(i):
                    r = i // VPR
                    cs = (i % VPR) * LANES
                    o[r, pl.ds(cs, LANES)] = (
                        g[r, pl.ds(cs, LANES)] * scale
                        + pos_v[p_off + r, pl.ds(cs, LANES)])

                pltpu.async_copy(
                    o, out_hbm.at[pl.ds(b * S + s_base + s_off, CH), :],
                    wsem[buf])

                @pl.when(t + 2 < T)
                def _():
                    start_gather(t + 2, buf)

        for buf in range(2):
            pltpu.make_async_copy(obuf[buf], out_hbm.at[pl.ds(0, CH), :],
                                  wsem[buf]).wait()

    return k


@jax.jit
def kernel(x, tok_emb, pos_emb):
    B, S = x.shape
    D = tok_emb.shape[1]
    xf = x.reshape(B * S).astype(jnp.int32)
    out = _make_kernel(B, S, D, CH=16)(xf, tok_emb, pos_emb)
    return out.reshape(B, S, D)
